# speedup vs baseline: 6.7541x; 6.7541x over previous
import functools

import jax
import jax.numpy as jnp
from jax.experimental import pallas as pl
from jax.experimental.pallas import tpu as pltpu

N_KENN_LAYERS = 3


def _mlp_body(x_ref, w1_ref, b1_ref, w2_ref, b2_ref, o_ref):
    h = jnp.maximum(
        jnp.dot(x_ref[...], w1_ref[...], preferred_element_type=jnp.float32)
        + b1_ref[...],
        0.0,
    )
    o_ref[...] = (
        jnp.dot(h, w2_ref[...], preferred_element_type=jnp.float32) + b2_ref[...]
    )


def _mlp_preact(features, W1, b1, W2, b2):
    n, d = features.shape
    blk = 1000
    grid = n // blk
    return pl.pallas_call(
        _mlp_body,
        grid=(grid,),
        in_specs=[
            pl.BlockSpec((blk, d), lambda i: (i, 0)),
            pl.BlockSpec((d, W1.shape[1]), lambda i: (0, 0)),
            pl.BlockSpec((1, W1.shape[1]), lambda i: (0, 0)),
            pl.BlockSpec((W1.shape[1], W2.shape[1]), lambda i: (0, 0)),
            pl.BlockSpec((1, W2.shape[1]), lambda i: (0, 0)),
        ],
        out_specs=pl.BlockSpec((blk, W2.shape[1]), lambda i: (i, 0)),
        out_shape=jax.ShapeDtypeStruct((n, W2.shape[1]), jnp.float32),
        compiler_params=pltpu.CompilerParams(
            dimension_semantics=("arbitrary",),
        ),
    )(features, W1, b1.reshape(1, -1), W2, b2.reshape(1, -1))


def _kenn_unary(xy, yz, xz, w):
    out_xy, out_yz, out_xz = [], [], []
    for c in range(3):
        a, b, dd = -xy[:, c], -yz[:, c], xz[:, c]
        win0 = (a >= b) & (a >= dd)
        win1 = (~win0) & (b >= dd)
        win2 = ~(win0 | win1)
        out_xy.append(xy[:, c] - jnp.where(win0, w[c], 0.0))
        out_yz.append(yz[:, c] - jnp.where(win1, w[c], 0.0))
        out_xz.append(xz[:, c] + jnp.where(win2, w[c], 0.0))
    return (jnp.stack(out_xy, 1), jnp.stack(out_yz, 1), jnp.stack(out_xz, 1))


def kernel(features, within_preactivations, index_xy, index_yz, index_xz,
           W1, b1, W2, b2, clause_weights):
    M = within_preactivations.shape[0]
    N = features.shape[0]
    E = index_xz.shape[0]

    across = _mlp_preact(features, W1, b1, W2, b2)
    pre = jnp.concatenate([within_preactivations, across], axis=0)

    winner = jnp.full((N,), -1, jnp.int32).at[
        jnp.where(index_xz >= M, index_xz - M, N)
    ].max(jnp.arange(E, dtype=jnp.int32), mode="drop")
    has = winner >= 0
    e = jnp.maximum(winner, 0)

    xy = pre[index_xy[e]]
    yz = pre[index_yz[e]]
    xz = across

    w = jax.nn.softplus(clause_weights)
    for l in range(N_KENN_LAYERS):
        xy, yz, xz = _kenn_unary(xy, yz, xz, w[l])

    out = jnp.where(has[:, None], xz, across)
    return (out, jax.nn.softmax(out, axis=1))


# speedup vs baseline: 51.6918x; 7.6534x over previous
import functools

import jax
import jax.numpy as jnp
from jax import lax
from jax.experimental import pallas as pl
from jax.experimental.pallas import tpu as pltpu
from jax.experimental.pallas import tpu_sc as plsc

N_KENN_LAYERS = 3
NC = 2
NS = 16
NW = NC * NS
L = 16


def _mlp_body(x_ref, w1_ref, b1_ref, w2_ref, b2_ref, o_ref):
    h = jnp.maximum(
        jnp.dot(x_ref[...], w1_ref[...], preferred_element_type=jnp.float32)
        + b1_ref[...],
        0.0,
    )
    o_ref[...] = (
        jnp.dot(h, w2_ref[...], preferred_element_type=jnp.float32) + b2_ref[...]
    )


def _mlp_preact(features, W1, b1, W2, b2):
    n, d = features.shape
    blk = 1000
    grid = n // blk
    return pl.pallas_call(
        _mlp_body,
        grid=(grid,),
        in_specs=[
            pl.BlockSpec((blk, d), lambda i: (i, 0)),
            pl.BlockSpec((d, W1.shape[1]), lambda i: (0, 0)),
            pl.BlockSpec((1, W1.shape[1]), lambda i: (0, 0)),
            pl.BlockSpec((W1.shape[1], W2.shape[1]), lambda i: (0, 0)),
            pl.BlockSpec((1, W2.shape[1]), lambda i: (0, 0)),
        ],
        out_specs=pl.BlockSpec((blk, W2.shape[1]), lambda i: (i, 0)),
        out_shape=jax.ShapeDtypeStruct((n, W2.shape[1]), jnp.float32),
        compiler_params=pltpu.CompilerParams(
            dimension_semantics=("arbitrary",),
        ),
    )(features, W1, b1.reshape(1, -1), W2, b2.reshape(1, -1))


def _make_winner_kernel(E, M, N):
    EPW = E // NW
    mesh = plsc.VectorSubcoreMesh(
        core_axis_name="c", subcore_axis_name="s", num_cores=NC, num_subcores=NS
    )

    @functools.partial(
        pl.kernel,
        out_type=jax.ShapeDtypeStruct((NW * N,), jnp.int32),
        mesh=mesh,
        compiler_params=pltpu.CompilerParams(needs_layout_passes=False),
        scratch_types=[
            pltpu.VMEM((EPW,), jnp.int32),
            pltpu.VMEM((N,), jnp.int32),
            pltpu.SemaphoreType.DMA,
        ],
    )
    def winner_kernel(idxxz_hbm, win_hbm, idx_v, win_v, sem):
        wid = lax.axis_index("s") * NC + lax.axis_index("c")
        pltpu.async_copy(idxxz_hbm.at[pl.ds(wid * EPW, EPW)], idx_v, sem).wait()

        neg1 = jnp.full((L,), -1, jnp.int32)

        def init_body(i, _):
            win_v[pl.ds(i * L, L)] = neg1
            return 0

        lax.fori_loop(0, N // L, init_body, 0)

        lanes = lax.iota(jnp.int32, L)
        ebase = wid * EPW

        def scan_body(i, _):
            v = idx_v[pl.ds(i * L, L)]
            msk = v >= M
            rows = jnp.where(msk, v - M, 0)
            evec = (ebase + i * L) + lanes

            def w_cond(b):
                return jnp.max(b) > 0

            def w_body(b):
                plsc.store_scatter(win_v, [rows], evec, mask=b != 0)
                g = plsc.load_gather(win_v, [rows], mask=msk)
                return (msk & (g < evec)).astype(jnp.int32)

            lax.while_loop(w_cond, w_body, msk.astype(jnp.int32))
            return 0

        lax.fori_loop(0, EPW // L, scan_body, 0)
        pltpu.sync_copy(win_v, win_hbm.at[pl.ds(wid * N, N)])

    return winner_kernel


def _make_kenn_kernel(E, M, N):
    RPW = 384
    NCH = RPW // 128
    LAST = ((N - RPW) // 8) * 8
    mesh = plsc.VectorSubcoreMesh(
        core_axis_name="c", subcore_axis_name="s", num_cores=NC, num_subcores=NS
    )

    @functools.partial(
        pl.kernel,
        out_type=(
            jax.ShapeDtypeStruct((N * 3,), jnp.float32),
            jax.ShapeDtypeStruct((N * 3,), jnp.float32),
        ),
        mesh=mesh,
        compiler_params=pltpu.CompilerParams(needs_layout_passes=False),
        scratch_types=[
            pltpu.VMEM((NW * RPW,), jnp.int32),
            pltpu.VMEM((RPW,), jnp.int32),
            pltpu.VMEM((RPW,), jnp.int32),
            pltpu.VMEM((RPW,), jnp.int32),
            pltpu.VMEM((RPW,), jnp.int32),
            pltpu.VMEM((RPW,), jnp.float32),
            pltpu.VMEM((RPW,), jnp.float32),
            pltpu.VMEM((RPW,), jnp.float32),
            pltpu.VMEM((RPW,), jnp.float32),
            pltpu.VMEM((RPW,), jnp.float32),
            pltpu.VMEM((RPW,), jnp.float32),
            pltpu.VMEM((RPW,), jnp.float32),
            pltpu.VMEM((RPW,), jnp.float32),
            pltpu.VMEM((RPW,), jnp.float32),
            pltpu.VMEM((RPW * 3,), jnp.float32),
            pltpu.VMEM((RPW * 3,), jnp.float32),
            pltpu.VMEM((L,), jnp.float32),
            pltpu.SemaphoreType.DMA,
            pltpu.SemaphoreType.DMA,
        ],
    )
    def kenn_kernel(win_hbm, ixy_hbm, iyz_hbm, p0_hbm, p1_hbm, p2_hbm, w_hbm,
                    out_hbm, soft_hbm,
                    winb, winv, ebuf, gxy, gyz,
                    xy0, xy1, xy2, yz0, yz1, yz2, xz0, xz1, xz2,
                    obf, sbf, wv, sem1, sem2):
        wid = lax.axis_index("s") * NC + lax.axis_index("c")
        base = jnp.minimum(wid * RPW, LAST)
        pcols = (p0_hbm, p1_hbm, p2_hbm)
        xys = (xy0, xy1, xy2)
        yzs = (yz0, yz1, yz2)
        xzs = (xz0, xz1, xz2)

        ds0 = [
            pltpu.async_copy(win_hbm.at[pl.ds(t * N + base, RPW)],
                             winb.at[pl.ds(t * RPW, RPW)], sem1)
            for t in range(NW)
        ]
        for c in range(3):
            ds0.append(pltpu.async_copy(pcols[c].at[pl.ds(M + base, RPW)],
                                        xzs[c], sem1))
        ds0.append(pltpu.async_copy(w_hbm, wv, sem1))
        for d in ds0:
            d.wait()

        def red_body(i, _):
            acc = jnp.full((L,), -1, jnp.int32)
            for t in range(NW):
                acc = jnp.maximum(acc, winb[pl.ds(t * RPW + i * L, L)])
            winv[pl.ds(i * L, L)] = acc
            ebuf[pl.ds(i * L, L)] = jnp.maximum(acc, 0)
            return 0

        lax.fori_loop(0, RPW // L, red_body, 0)

        ds1 = []
        for j in range(NCH):
            sl = pl.ds(j * 128, 128)
            ds1.append(pltpu.async_copy(ixy_hbm.at[ebuf.at[sl]], gxy.at[sl], sem1))
            ds1.append(pltpu.async_copy(iyz_hbm.at[ebuf.at[sl]], gyz.at[sl], sem1))
        for d in ds1:
            d.wait()
        ds2 = []
        for j in range(NCH):
            sl = pl.ds(j * 128, 128)
            for c in range(3):
                ds2.append(pltpu.async_copy(pcols[c].at[gxy.at[sl]],
                                            xys[c].at[sl], sem2))
                ds2.append(pltpu.async_copy(pcols[c].at[gyz.at[sl]],
                                            yzs[c].at[sl], sem2))
        for d in ds2:
            d.wait()

        wvec = wv[...]
        ws = [wvec[k] for k in range(N_KENN_LAYERS * 3)]
        lanes = lax.iota(jnp.int32, L)

        def compute_body(i, _):
            sl = pl.ds(i * L, L)
            rows16 = i * L + lanes
            has = winv[sl] >= 0
            xy = [xys[c][sl] for c in range(3)]
            yz = [yzs[c][sl] for c in range(3)]
            xz = [xzs[c][sl] for c in range(3)]
            xz0_ = list(xz)
            for l in range(N_KENN_LAYERS):
                for c in range(3):
                    a, b, dd = -xy[c], -yz[c], xz[c]
                    w = ws[l * 3 + c]
                    win0 = (a >= b) & (a >= dd)
                    win1 = (~win0) & (b >= dd)
                    win2 = ~(win0 | win1)
                    xy[c] = xy[c] - jnp.where(win0, w, 0.0)
                    yz[c] = yz[c] - jnp.where(win1, w, 0.0)
                    xz[c] = xz[c] + jnp.where(win2, w, 0.0)
            o = [jnp.where(has, xz[c], xz0_[c]) for c in range(3)]
            m = jnp.maximum(jnp.maximum(o[0], o[1]), o[2])
            ex = [jnp.exp(o[c] - m) for c in range(3)]
            ssum = ex[0] + ex[1] + ex[2]
            for c in range(3):
                sidx = rows16 * 3 + c
                plsc.store_scatter(obf, [sidx], o[c])
                plsc.store_scatter(sbf, [sidx], ex[c] / ssum)
            return 0

        lax.fori_loop(0, RPW // L, compute_body, 0)

        pltpu.sync_copy(obf, out_hbm.at[pl.ds(base * 3, RPW * 3)])
        pltpu.sync_copy(sbf, soft_hbm.at[pl.ds(base * 3, RPW * 3)])

    return kenn_kernel


def kernel(features, within_preactivations, index_xy, index_yz, index_xz,
           W1, b1, W2, b2, clause_weights):
    M = within_preactivations.shape[0]
    N = features.shape[0]
    E = index_xz.shape[0]

    across = _mlp_preact(features, W1, b1, W2, b2)
    preT = jnp.concatenate([within_preactivations, across], axis=0).T

    wsp = jax.nn.softplus(clause_weights).reshape(-1)
    wsp16 = jnp.zeros((L,), jnp.float32).at[: wsp.shape[0]].set(wsp)

    winners = _make_winner_kernel(E, M, N)(index_xz)
    outf, softf = _make_kenn_kernel(E, M, N)(
        winners, index_xy, index_yz, preT[0], preT[1], preT[2], wsp16
    )
    return (outf.reshape(N, 3), softf.reshape(N, 3))


# speedup vs baseline: 54.8936x; 1.0619x over previous
import functools

import jax
import jax.numpy as jnp
from jax import lax
from jax.experimental import pallas as pl
from jax.experimental.pallas import tpu as pltpu
from jax.experimental.pallas import tpu_sc as plsc

N_KENN_LAYERS = 3
NC = 2
NS = 16
NW = NC * NS
L = 16


def _mlp_body(x_ref, w1_ref, b1_ref, w2_ref, b2_ref, o_ref):
    h = jnp.maximum(
        jnp.dot(x_ref[...], w1_ref[...], preferred_element_type=jnp.float32)
        + b1_ref[...],
        0.0,
    )
    o_ref[...] = (
        jnp.dot(h, w2_ref[...], preferred_element_type=jnp.float32) + b2_ref[...]
    )


def _mlp_preact(features, W1, b1, W2, b2):
    n, d = features.shape
    blk = 1000
    grid = n // blk
    return pl.pallas_call(
        _mlp_body,
        grid=(grid,),
        in_specs=[
            pl.BlockSpec((blk, d), lambda i: (i, 0)),
            pl.BlockSpec((d, W1.shape[1]), lambda i: (0, 0)),
            pl.BlockSpec((1, W1.shape[1]), lambda i: (0, 0)),
            pl.BlockSpec((W1.shape[1], W2.shape[1]), lambda i: (0, 0)),
            pl.BlockSpec((1, W2.shape[1]), lambda i: (0, 0)),
        ],
        out_specs=pl.BlockSpec((blk, W2.shape[1]), lambda i: (i, 0)),
        out_shape=jax.ShapeDtypeStruct((n, W2.shape[1]), jnp.float32),
        compiler_params=pltpu.CompilerParams(
            dimension_semantics=("arbitrary",),
        ),
    )(features, W1, b1.reshape(1, -1), W2, b2.reshape(1, -1))


def _make_winner_kernel(E, M, N):
    EPW = E // NW
    mesh = plsc.VectorSubcoreMesh(
        core_axis_name="c", subcore_axis_name="s", num_cores=NC, num_subcores=NS
    )

    @functools.partial(
        pl.kernel,
        out_type=jax.ShapeDtypeStruct((NW * N,), jnp.int32),
        mesh=mesh,
        compiler_params=pltpu.CompilerParams(needs_layout_passes=False),
        scratch_types=[
            pltpu.VMEM((EPW,), jnp.int32),
            pltpu.VMEM((N,), jnp.int32),
            pltpu.SemaphoreType.DMA,
        ],
    )
    def winner_kernel(idxxz_hbm, win_hbm, idx_v, win_v, sem):
        wid = lax.axis_index("s") * NC + lax.axis_index("c")
        pltpu.async_copy(idxxz_hbm.at[pl.ds(wid * EPW, EPW)], idx_v, sem).wait()

        neg1 = jnp.full((L,), -1, jnp.int32)

        def init_body(i, _):
            win_v[pl.ds(i * L, L)] = neg1
            return 0

        lax.fori_loop(0, N // L, init_body, 0)

        lanes = lax.iota(jnp.int32, L)
        ebase = wid * EPW

        def scan_body(i, _):
            v = idx_v[pl.ds(i * L, L)]
            msk = v >= M
            rows = jnp.where(msk, v - M, 0)
            evec = (ebase + i * L) + lanes

            def w_cond(b):
                return jnp.max(b) > 0

            def w_body(b):
                plsc.store_scatter(win_v, [rows], evec, mask=b != 0)
                g = plsc.load_gather(win_v, [rows], mask=msk)
                return (msk & (g < evec)).astype(jnp.int32)

            lax.while_loop(w_cond, w_body, msk.astype(jnp.int32))
            return 0

        lax.fori_loop(0, EPW // L, scan_body, 0)
        pltpu.sync_copy(win_v, win_hbm.at[pl.ds(wid * N, N)])

    return winner_kernel


def _make_kenn_kernel(E, M, N):
    RPW = 384
    NCH = RPW // 128
    LAST = ((N - RPW) // 8) * 8
    mesh = plsc.VectorSubcoreMesh(
        core_axis_name="c", subcore_axis_name="s", num_cores=NC, num_subcores=NS
    )

    @functools.partial(
        pl.kernel,
        out_type=(
            jax.ShapeDtypeStruct((N, 3), jnp.float32),
            jax.ShapeDtypeStruct((N, 3), jnp.float32),
        ),
        mesh=mesh,
        compiler_params=pltpu.CompilerParams(needs_layout_passes=False),
        scratch_types=[
            pltpu.VMEM((NW * RPW,), jnp.int32),
            pltpu.VMEM((RPW,), jnp.int32),
            pltpu.VMEM((RPW,), jnp.int32),
            pltpu.VMEM((RPW,), jnp.int32),
            pltpu.VMEM((RPW,), jnp.int32),
            pltpu.VMEM((RPW,), jnp.float32),
            pltpu.VMEM((RPW,), jnp.float32),
            pltpu.VMEM((RPW,), jnp.float32),
            pltpu.VMEM((RPW,), jnp.float32),
            pltpu.VMEM((RPW,), jnp.float32),
            pltpu.VMEM((RPW,), jnp.float32),
            pltpu.VMEM((RPW,), jnp.float32),
            pltpu.VMEM((RPW,), jnp.float32),
            pltpu.VMEM((RPW,), jnp.float32),
            pltpu.VMEM((RPW, 3), jnp.float32),
            pltpu.VMEM((RPW, 3), jnp.float32),
            pltpu.VMEM((L,), jnp.float32),
            pltpu.SemaphoreType.DMA,
            pltpu.SemaphoreType.DMA,
        ],
    )
    def kenn_kernel(win_hbm, ixy_hbm, iyz_hbm, p0_hbm, p1_hbm, p2_hbm, w_hbm,
                    out_hbm, soft_hbm,
                    winb, winv, ebuf, gxy, gyz,
                    xy0, xy1, xy2, yz0, yz1, yz2, xz0, xz1, xz2,
                    obf, sbf, wv, sem1, sem2):
        wid = lax.axis_index("s") * NC + lax.axis_index("c")
        base = jnp.minimum(wid * RPW, LAST)
        pcols = (p0_hbm, p1_hbm, p2_hbm)
        xys = (xy0, xy1, xy2)
        yzs = (yz0, yz1, yz2)
        xzs = (xz0, xz1, xz2)

        ds0 = [
            pltpu.async_copy(win_hbm.at[pl.ds(t * N + base, RPW)],
                             winb.at[pl.ds(t * RPW, RPW)], sem1)
            for t in range(NW)
        ]
        for c in range(3):
            ds0.append(pltpu.async_copy(pcols[c].at[pl.ds(M + base, RPW)],
                                        xzs[c], sem1))
        ds0.append(pltpu.async_copy(w_hbm, wv, sem1))
        for d in ds0:
            d.wait()

        def red_body(i, _):
            acc = jnp.full((L,), -1, jnp.int32)
            for t in range(NW):
                acc = jnp.maximum(acc, winb[pl.ds(t * RPW + i * L, L)])
            winv[pl.ds(i * L, L)] = acc
            ebuf[pl.ds(i * L, L)] = jnp.maximum(acc, 0)
            return 0

        lax.fori_loop(0, RPW // L, red_body, 0)

        ds1 = []
        for j in range(NCH):
            sl = pl.ds(j * 128, 128)
            ds1.append(pltpu.async_copy(ixy_hbm.at[ebuf.at[sl]], gxy.at[sl], sem1))
            ds1.append(pltpu.async_copy(iyz_hbm.at[ebuf.at[sl]], gyz.at[sl], sem1))
        for d in ds1:
            d.wait()
        ds2 = []
        for j in range(NCH):
            sl = pl.ds(j * 128, 128)
            for c in range(3):
                ds2.append(pltpu.async_copy(pcols[c].at[gxy.at[sl]],
                                            xys[c].at[sl], sem2))
                ds2.append(pltpu.async_copy(pcols[c].at[gyz.at[sl]],
                                            yzs[c].at[sl], sem2))
        for d in ds2:
            d.wait()

        wvec = wv[...]
        ws = [wvec[k] for k in range(N_KENN_LAYERS * 3)]
        lanes = lax.iota(jnp.int32, L)

        def compute_body(i, _):
            sl = pl.ds(i * L, L)
            rows16 = i * L + lanes
            has = winv[sl] >= 0
            xy = [xys[c][sl] for c in range(3)]
            yz = [yzs[c][sl] for c in range(3)]
            xz = [xzs[c][sl] for c in range(3)]
            xz0_ = list(xz)
            for l in range(N_KENN_LAYERS):
                for c in range(3):
                    a, b, dd = -xy[c], -yz[c], xz[c]
                    w = ws[l * 3 + c]
                    win0 = (a >= b) & (a >= dd)
                    win1 = (~win0) & (b >= dd)
                    win2 = ~(win0 | win1)
                    xy[c] = xy[c] - jnp.where(win0, w, 0.0)
                    yz[c] = yz[c] - jnp.where(win1, w, 0.0)
                    xz[c] = xz[c] + jnp.where(win2, w, 0.0)
            o = [jnp.where(has, xz[c], xz0_[c]) for c in range(3)]
            m = jnp.maximum(jnp.maximum(o[0], o[1]), o[2])
            ex = [jnp.exp(o[c] - m) for c in range(3)]
            ssum = ex[0] + ex[1] + ex[2]
            for c in range(3):
                cvec = jnp.full((L,), c, jnp.int32)
                plsc.store_scatter(obf, [rows16, cvec], o[c])
                plsc.store_scatter(sbf, [rows16, cvec], ex[c] / ssum)
            return 0

        lax.fori_loop(0, RPW // L, compute_body, 0)

        pltpu.sync_copy(obf, out_hbm.at[pl.ds(base, RPW), :])
        pltpu.sync_copy(sbf, soft_hbm.at[pl.ds(base, RPW), :])

    return kenn_kernel


def kernel(features, within_preactivations, index_xy, index_yz, index_xz,
           W1, b1, W2, b2, clause_weights):
    M = within_preactivations.shape[0]
    N = features.shape[0]
    E = index_xz.shape[0]

    across = _mlp_preact(features, W1, b1, W2, b2)
    preT = jnp.concatenate([within_preactivations, across], axis=0).T

    wsp = jax.nn.softplus(clause_weights).reshape(-1)
    wsp16 = jnp.zeros((L,), jnp.float32).at[: wsp.shape[0]].set(wsp)

    winners = _make_winner_kernel(E, M, N)(index_xz)
    out, soft = _make_kenn_kernel(E, M, N)(
        winners, index_xy, index_yz, preT[0], preT[1], preT[2], wsp16
    )
    return (out, soft)


# speedup vs baseline: 65.8868x; 1.2003x over previous
import functools

import jax
import jax.numpy as jnp
from jax import lax
from jax.experimental import pallas as pl
from jax.experimental.pallas import tpu as pltpu
from jax.experimental.pallas import tpu_sc as plsc

N_KENN_LAYERS = 3
NC = 2
NS = 16
NW = NC * NS
L = 16


def _mlp_body(x_ref, w1_ref, b1_ref, w2_ref, b2_ref, o_ref):
    h = jnp.maximum(
        jnp.dot(x_ref[...], w1_ref[...], preferred_element_type=jnp.float32)
        + b1_ref[...],
        0.0,
    )
    o_ref[...] = (
        jnp.dot(h, w2_ref[...], preferred_element_type=jnp.float32) + b2_ref[...]
    )


def _mlp_preact(features, W1, b1, W2, b2):
    n, d = features.shape
    blk = 1000
    grid = n // blk
    return pl.pallas_call(
        _mlp_body,
        grid=(grid,),
        in_specs=[
            pl.BlockSpec((blk, d), lambda i: (i, 0)),
            pl.BlockSpec((d, W1.shape[1]), lambda i: (0, 0)),
            pl.BlockSpec((1, W1.shape[1]), lambda i: (0, 0)),
            pl.BlockSpec((W1.shape[1], W2.shape[1]), lambda i: (0, 0)),
            pl.BlockSpec((1, W2.shape[1]), lambda i: (0, 0)),
        ],
        out_specs=pl.BlockSpec((blk, W2.shape[1]), lambda i: (i, 0)),
        out_shape=jax.ShapeDtypeStruct((n, W2.shape[1]), jnp.float32),
        compiler_params=pltpu.CompilerParams(
            dimension_semantics=("arbitrary",),
        ),
    )(features, W1, b1.reshape(1, -1), W2, b2.reshape(1, -1))


def _make_winner_kernel(E, M, N):
    EPW = E // NW
    mesh = plsc.VectorSubcoreMesh(
        core_axis_name="c", subcore_axis_name="s", num_cores=NC, num_subcores=NS
    )

    @functools.partial(
        pl.kernel,
        out_type=jax.ShapeDtypeStruct((NW * N,), jnp.int32),
        mesh=mesh,
        compiler_params=pltpu.CompilerParams(needs_layout_passes=False),
        scratch_types=[
            pltpu.VMEM((EPW,), jnp.int32),
            pltpu.VMEM((N,), jnp.int32),
            pltpu.VMEM((EPW + L,), jnp.int32),
            pltpu.VMEM((EPW + L,), jnp.int32),
            pltpu.SemaphoreType.DMA,
        ],
    )
    def winner_kernel(idxxz_hbm, win_hbm, idx_v, win_v, rowc, evc, sem):
        wid = lax.axis_index("s") * NC + lax.axis_index("c")
        pltpu.async_copy(idxxz_hbm.at[pl.ds(wid * EPW, EPW)], idx_v, sem).wait()

        neg1 = jnp.full((L,), -1, jnp.int32)

        def init_body(i, _):
            win_v[pl.ds(i * L, L)] = neg1
            return 0

        lax.fori_loop(0, N // L, init_body, 0)

        lanes = lax.iota(jnp.int32, L)
        ebase = wid * EPW

        def compact_body(i, off):
            v = idx_v[pl.ds(i * L, L)]
            msk = v >= M
            evec = (ebase + i * L) + lanes
            plsc.store_compressed(rowc.at[pl.ds(off, L)], v - M, mask=msk)
            plsc.store_compressed(evc.at[pl.ds(off, L)], evec, mask=msk)
            return off + plsc.all_reduce_population_count(msk)[0]

        total = lax.fori_loop(0, EPW // L, compact_body, 0)

        def settle_body(j, _):
            rows = rowc[pl.ds(j * L, L)]
            evec = evc[pl.ds(j * L, L)]
            lanemask = (j * L + lanes) < total

            def w_cond(b):
                return jnp.max(b) > 0

            def w_body(b):
                plsc.store_scatter(win_v, [rows], evec, mask=b != 0)
                g = plsc.load_gather(win_v, [rows], mask=lanemask)
                return (lanemask & (g < evec)).astype(jnp.int32)

            lax.while_loop(w_cond, w_body, lanemask.astype(jnp.int32))
            return 0

        lax.fori_loop(0, (total + L - 1) // L, settle_body, 0)
        pltpu.sync_copy(win_v, win_hbm.at[pl.ds(wid * N, N)])

    return winner_kernel


def _make_kenn_kernel(E, M, N):
    RPW = 384
    NCH = RPW // 128
    LAST = ((N - RPW) // 8) * 8
    mesh = plsc.VectorSubcoreMesh(
        core_axis_name="c", subcore_axis_name="s", num_cores=NC, num_subcores=NS
    )

    @functools.partial(
        pl.kernel,
        out_type=(
            jax.ShapeDtypeStruct((N, 3), jnp.float32),
            jax.ShapeDtypeStruct((N, 3), jnp.float32),
        ),
        mesh=mesh,
        compiler_params=pltpu.CompilerParams(needs_layout_passes=False),
        scratch_types=[
            pltpu.VMEM((NW * RPW,), jnp.int32),
            pltpu.VMEM((RPW,), jnp.int32),
            pltpu.VMEM((RPW,), jnp.int32),
            pltpu.VMEM((RPW,), jnp.int32),
            pltpu.VMEM((RPW,), jnp.int32),
            pltpu.VMEM((RPW,), jnp.float32),
            pltpu.VMEM((RPW,), jnp.float32),
            pltpu.VMEM((RPW,), jnp.float32),
            pltpu.VMEM((RPW,), jnp.float32),
            pltpu.VMEM((RPW,), jnp.float32),
            pltpu.VMEM((RPW,), jnp.float32),
            pltpu.VMEM((RPW,), jnp.float32),
            pltpu.VMEM((RPW,), jnp.float32),
            pltpu.VMEM((RPW,), jnp.float32),
            pltpu.VMEM((RPW, 3), jnp.float32),
            pltpu.VMEM((RPW, 3), jnp.float32),
            pltpu.VMEM((L,), jnp.float32),
            pltpu.SemaphoreType.DMA,
            pltpu.SemaphoreType.DMA,
        ],
    )
    def kenn_kernel(win_hbm, ixy_hbm, iyz_hbm, p0_hbm, p1_hbm, p2_hbm, w_hbm,
                    out_hbm, soft_hbm,
                    winb, winv, ebuf, gxy, gyz,
                    xy0, xy1, xy2, yz0, yz1, yz2, xz0, xz1, xz2,
                    obf, sbf, wv, sem1, sem2):
        wid = lax.axis_index("s") * NC + lax.axis_index("c")
        base = jnp.minimum(wid * RPW, LAST)
        pcols = (p0_hbm, p1_hbm, p2_hbm)
        xys = (xy0, xy1, xy2)
        yzs = (yz0, yz1, yz2)
        xzs = (xz0, xz1, xz2)

        ds0 = [
            pltpu.async_copy(win_hbm.at[pl.ds(t * N + base, RPW)],
                             winb.at[pl.ds(t * RPW, RPW)], sem1)
            for t in range(NW)
        ]
        for c in range(3):
            ds0.append(pltpu.async_copy(pcols[c].at[pl.ds(M + base, RPW)],
                                        xzs[c], sem1))
        ds0.append(pltpu.async_copy(w_hbm, wv, sem1))
        for d in ds0:
            d.wait()

        def red_body(i, _):
            acc = jnp.full((L,), -1, jnp.int32)
            for t in range(NW):
                acc = jnp.maximum(acc, winb[pl.ds(t * RPW + i * L, L)])
            winv[pl.ds(i * L, L)] = acc
            ebuf[pl.ds(i * L, L)] = jnp.maximum(acc, 0)
            return 0

        lax.fori_loop(0, RPW // L, red_body, 0)

        ds1 = []
        for j in range(NCH):
            sl = pl.ds(j * 128, 128)
            ds1.append(pltpu.async_copy(ixy_hbm.at[ebuf.at[sl]], gxy.at[sl], sem1))
            ds1.append(pltpu.async_copy(iyz_hbm.at[ebuf.at[sl]], gyz.at[sl], sem1))
        for d in ds1:
            d.wait()
        ds2 = []
        for j in range(NCH):
            sl = pl.ds(j * 128, 128)
            for c in range(3):
                ds2.append(pltpu.async_copy(pcols[c].at[gxy.at[sl]],
                                            xys[c].at[sl], sem2))
                ds2.append(pltpu.async_copy(pcols[c].at[gyz.at[sl]],
                                            yzs[c].at[sl], sem2))
        for d in ds2:
            d.wait()

        wvec = wv[...]
        ws = [wvec[k] for k in range(N_KENN_LAYERS * 3)]
        lanes = lax.iota(jnp.int32, L)

        def compute_body(i, _):
            sl = pl.ds(i * L, L)
            rows16 = i * L + lanes
            has = winv[sl] >= 0
            xy = [xys[c][sl] for c in range(3)]
            yz = [yzs[c][sl] for c in range(3)]
            xz = [xzs[c][sl] for c in range(3)]
            xz0_ = list(xz)
            for l in range(N_KENN_LAYERS):
                for c in range(3):
                    a, b, dd = -xy[c], -yz[c], xz[c]
                    w = ws[l * 3 + c]
                    win0 = (a >= b) & (a >= dd)
                    win1 = (~win0) & (b >= dd)
                    win2 = ~(win0 | win1)
                    xy[c] = xy[c] - jnp.where(win0, w, 0.0)
                    yz[c] = yz[c] - jnp.where(win1, w, 0.0)
                    xz[c] = xz[c] + jnp.where(win2, w, 0.0)
            o = [jnp.where(has, xz[c], xz0_[c]) for c in range(3)]
            m = jnp.maximum(jnp.maximum(o[0], o[1]), o[2])
            ex = [jnp.exp(o[c] - m) for c in range(3)]
            ssum = ex[0] + ex[1] + ex[2]
            for c in range(3):
                cvec = jnp.full((L,), c, jnp.int32)
                plsc.store_scatter(obf, [rows16, cvec], o[c])
                plsc.store_scatter(sbf, [rows16, cvec], ex[c] / ssum)
            return 0

        lax.fori_loop(0, RPW // L, compute_body, 0)

        pltpu.sync_copy(obf, out_hbm.at[pl.ds(base, RPW), :])
        pltpu.sync_copy(sbf, soft_hbm.at[pl.ds(base, RPW), :])

    return kenn_kernel


def kernel(features, within_preactivations, index_xy, index_yz, index_xz,
           W1, b1, W2, b2, clause_weights):
    M = within_preactivations.shape[0]
    N = features.shape[0]
    E = index_xz.shape[0]

    across = _mlp_preact(features, W1, b1, W2, b2)
    preT = jnp.concatenate([within_preactivations, across], axis=0).T

    wsp = jax.nn.softplus(clause_weights).reshape(-1)
    wsp16 = jnp.zeros((L,), jnp.float32).at[: wsp.shape[0]].set(wsp)

    winners = _make_winner_kernel(E, M, N)(index_xz)
    out, soft = _make_kenn_kernel(E, M, N)(
        winners, index_xy, index_yz, preT[0], preT[1], preT[2], wsp16
    )
    return (out, soft)
